# baseline (speedup 1.0000x reference)
import jax
import jax.numpy as jnp
from jax.experimental import pallas as pl

_P = 21824
_G = 500
_ROW_BLK = 512
_G_PAD = 512
_W = 125


def _body(m32_ref, out_ref):
    s = m32_ref[...].astype(jnp.float32)
    r = jnp.sum(s, axis=1, keepdims=True)
    out_ref[...] = jnp.broadcast_to(r, (_ROW_BLK, _G_PAD))


def kernel(points0, points1, points2, points3, points4,
           gt_bboxes, labels, inside_gt_bbox_mask, mean, sigma):
    m32 = inside_gt_bbox_mask.view(jnp.int32)
    w = pl.pallas_call(
        _body,
        grid=(pl.cdiv(_P, _ROW_BLK),),
        in_specs=[pl.BlockSpec((_ROW_BLK, _W), lambda i: (i, 0))],
        out_specs=pl.BlockSpec((_ROW_BLK, _G_PAD), lambda i: (i, 0)),
        out_shape=jax.ShapeDtypeStruct((_P, _G), jnp.float32),
    )(m32)
    return (w, inside_gt_bbox_mask)


# speedup vs baseline: 1.2675x; 1.2675x over previous
import functools

import jax
import jax.numpy as jnp
from jax.experimental import pallas as pl
from jax.experimental.pallas import tpu as pltpu

_STRIDES = (8.0, 16.0, 32.0, 64.0, 128.0)
_SIZES = (16384, 4096, 1024, 256, 64)
_P = sum(_SIZES)
_G = 500
_G_PAD = 512
_ROW_BLK = 512
_K = 16


def _body(pts_ref, gt_ref, lab_ref, mean_ref, sig_ref, mask_ref, out_ref, b_ref, s_ref):
    i = pl.program_id(0)

    @pl.when(i == 0)
    def _init():
        cx = (gt_ref[0:1, :] + gt_ref[2:3, :]) * 0.5
        cy = (gt_ref[1:2, :] + gt_ref[3:4, :]) * 0.5
        lab = lab_ref[0:1, :]
        cls = jax.lax.broadcasted_iota(jnp.int32, (128, _G_PAD), 0)
        oh = (jnp.broadcast_to(lab, (128, _G_PAD)) == cls).astype(jnp.float32)
        mx = jnp.sum(oh * mean_ref[:, 0:1], axis=0, keepdims=True)
        my = jnp.sum(oh * mean_ref[:, 1:2], axis=0, keepdims=True)
        sx = jnp.sum(oh * sig_ref[:, 0:1], axis=0, keepdims=True)
        sy = jnp.sum(oh * sig_ref[:, 1:2], axis=0, keepdims=True)
        one = jnp.ones((1, _G_PAD), jnp.float32)
        zero = jnp.zeros((1, _G_PAD), jnp.float32)
        b_ref[0:1, 0:_G_PAD] = one
        b_ref[1:2, 0:_G_PAD] = zero
        b_ref[2:3, 0:_G_PAD] = -cx
        b_ref[3:4, 0:_G_PAD] = -mx
        b_ref[4:8, 0:_G_PAD] = jnp.zeros((4, _G_PAD), jnp.float32)
        b_ref[0:1, _G_PAD:] = zero
        b_ref[1:2, _G_PAD:] = one
        b_ref[2:3, _G_PAD:] = -cy
        b_ref[3:4, _G_PAD:] = -my
        b_ref[4:8, _G_PAD:] = jnp.zeros((4, _G_PAD), jnp.float32)
        s_ref[0:1, 0:_G_PAD] = -0.5 / (sx * sx)
        s_ref[0:1, _G_PAD:] = -0.5 / (sy * sy)

    x = pts_ref[:, 0:1]
    y = pts_ref[:, 1:2]
    s = pts_ref[:, 2:3]
    u = x * s
    v = y * s
    lane = jax.lax.broadcasted_iota(jnp.int32, (_ROW_BLK, 8), 1)
    a = jnp.where(lane == 0, jnp.broadcast_to(u, (_ROW_BLK, 8)),
                  jnp.where(lane == 1, jnp.broadcast_to(v, (_ROW_BLK, 8)),
                            jnp.where(lane == 2, jnp.broadcast_to(s, (_ROW_BLK, 8)),
                                      jnp.where(lane == 3, 1.0, 0.0))))
    d = jax.lax.dot_general(
        a,
        b_ref[...],
        dimension_numbers=(((1,), (0,)), ((), ())),
        preferred_element_type=jnp.float32,
        precision=jax.lax.Precision.HIGHEST,
    )
    q = (d * d) * jnp.broadcast_to(s_ref[0:1, :], (_ROW_BLK, 2 * _G_PAD))
    t = q[:, 0:_G_PAD] + q[:, _G_PAD:]
    w = jnp.exp(t)
    out_ref[...] = jnp.where(mask_ref[...], w, 0.0)


@functools.partial(jax.jit, static_argnames=())
def _center_prior_tc(pts3, gt_t, lab_p, mean_p, sig_p, mask):
    grid = (pl.cdiv(_P, _ROW_BLK),)
    return pl.pallas_call(
        _body,
        grid=grid,
        in_specs=[
            pl.BlockSpec((_ROW_BLK, 4), lambda i: (i, 0)),
            pl.BlockSpec((8, _G_PAD), lambda i: (0, 0)),
            pl.BlockSpec((8, _G_PAD), lambda i: (0, 0)),
            pl.BlockSpec((128, 128), lambda i: (0, 0)),
            pl.BlockSpec((128, 128), lambda i: (0, 0)),
            pl.BlockSpec((_ROW_BLK, _G_PAD), lambda i: (i, 0)),
        ],
        out_specs=pl.BlockSpec((_ROW_BLK, _G_PAD), lambda i: (i, 0)),
        out_shape=jax.ShapeDtypeStruct((_P, _G), jnp.float32),
        scratch_shapes=[
            pltpu.VMEM((8, 2 * _G_PAD), jnp.float32),
            pltpu.VMEM((8, 2 * _G_PAD), jnp.float32),
        ],
        compiler_params=pltpu.CompilerParams(
            dimension_semantics=("arbitrary",),
        ),
    )(pts3, gt_t, lab_p, mean_p, sig_p, mask)


def kernel(points0, points1, points2, points3, points4,
           gt_bboxes, labels, inside_gt_bbox_mask, mean, sigma):
    pts = jnp.concatenate([points0, points1, points2, points3, points4], axis=0)
    inv_s = jnp.repeat(
        jnp.asarray([1.0 / s for s in _STRIDES], jnp.float32),
        jnp.asarray(_SIZES),
        total_repeat_length=_P,
    )
    pts3 = jnp.concatenate(
        [pts, inv_s[:, None], jnp.zeros((_P, 1), jnp.float32)], axis=1)

    gt_t = jnp.zeros((8, _G_PAD), jnp.float32).at[:4, :_G].set(gt_bboxes.T)
    lab_p = jnp.zeros((8, _G_PAD), jnp.int32).at[0, :_G].set(labels.astype(jnp.int32))
    mean_p = jnp.zeros((128, 128), jnp.float32).at[:80, :2].set(mean)
    sig_p = jnp.ones((128, 128), jnp.float32).at[:80, :2].set(sigma)

    w = _center_prior_tc(pts3, gt_t, lab_p, mean_p, sig_p, inside_gt_bbox_mask)
    return (w, inside_gt_bbox_mask)
